# baseline (speedup 1.0000x reference)
import dataclasses
import functools

import jax
import jax.numpy as jnp
from jax import lax
from jax.experimental import pallas as pl
from jax.experimental.pallas import tpu as pltpu
from jax.experimental.pallas import tpu_sc as plsc

NC = 2
NS = 16
NW = NC * NS
L = 16
NSLICE = 1


def kernel(xs, t, y):
    del t
    n = xs.shape[0]
    nk = y.shape[0]
    chunk = n // NW
    scale = jnp.float32(nk - 1)

    mesh = plsc.VectorSubcoreMesh(core_axis_name="c", subcore_axis_name="s")

    cp = pltpu.CompilerParams()
    if "needs_layout_passes" in pltpu.CompilerParams.__dataclass_fields__:
        cp = dataclasses.replace(cp, needs_layout_passes=False)

    @functools.partial(
        pl.kernel,
        out_type=jax.ShapeDtypeStruct((n,), jnp.float32),
        mesh=mesh,
        compiler_params=cp,
        scratch_types=[
            pltpu.VMEM((chunk,), jnp.float32),
            pltpu.VMEM((chunk,), jnp.float32),
            pltpu.VMEM((nk,), jnp.float32),
            pltpu.VMEM((nk,), jnp.float32),
            pltpu.VMEM((nk,), jnp.float32),
            pltpu.VMEM((nk,), jnp.int32),
            pltpu.VMEM((nk,), jnp.int32),
            pltpu.SemaphoreType.DMA((NSLICE,)),
            pltpu.SemaphoreType.DMA((NSLICE,)),
            pltpu.SemaphoreType.DMA,
        ],
    )
    def sc_spline(xs_hbm, y_hbm, out_hbm, xv, ov, yv, dv, gv, pv, qv,
                  in_sems, out_sems, ysem):
        def rne16(v):
            u = lax.bitcast_convert_type(v, jnp.uint32)
            return (u + jnp.uint32(0x7FFF) + ((u >> 16) & jnp.uint32(1))) >> 16

        def unpack_hi(w):
            return lax.bitcast_convert_type(w & jnp.uint32(0xFFFF0000),
                                            jnp.float32)

        def unpack_lo(w):
            return lax.bitcast_convert_type(w << 16, jnp.float32)

        wid = lax.axis_index("s") * NC + lax.axis_index("c")
        base = wid * chunk
        sl = chunk // NSLICE
        in_cpys = [
            pltpu.async_copy(
                xs_hbm.at[pl.ds(base + s * sl, sl)],
                xv.at[pl.ds(s * sl, sl)],
                in_sems.at[s],
            )
            for s in range(NSLICE)
        ]
        pltpu.async_copy(y_hbm, yv, ysem).wait()

        @plsc.parallel_loop(0, nk, step=L, unroll=4)
        def _(c):
            idx = lax.iota(jnp.int32, L) + c
            y0 = plsc.load_gather(yv, [jnp.minimum(idx, nk - 1)])
            y1 = plsc.load_gather(yv, [jnp.minimum(idx + 1, nk - 1)])
            dv[pl.ds(c, L)] = y1 - y0

        @plsc.parallel_loop(0, nk, step=L, unroll=4)
        def _(c):
            idx = lax.iota(jnp.int32, L) + c
            lo = plsc.load_gather(dv, [jnp.maximum(idx - 1, 0)])
            hi = plsc.load_gather(dv, [jnp.minimum(idx, nk - 2)])
            gv[pl.ds(c, L)] = 0.5 * (lo + hi)

        @plsc.parallel_loop(0, nk, step=L, unroll=4)
        def _(c):
            idx = lax.iota(jnp.int32, L) + c
            dd = plsc.load_gather(dv, [jnp.minimum(idx, nk - 2)])
            gi = plsc.load_gather(gv, [idx])
            gi1 = plsc.load_gather(gv, [jnp.minimum(idx + 1, nk - 1)])
            y16 = yv[pl.ds(c, L)]
            cc = 3.0 * dd - 2.0 * gi - gi1
            qq = gi + gi1 - 2.0 * dd
            pv[pl.ds(c, L)] = lax.bitcast_convert_type(
                (rne16(y16) << 16) | rne16(gi), jnp.int32)
            qv[pl.ds(c, L)] = lax.bitcast_convert_type(
                (rne16(cc) << 16) | rne16(qq), jnp.int32)

        out_cpys = []
        for s in range(NSLICE):
            in_cpys[s].wait()

            @plsc.parallel_loop(s * sl, (s + 1) * sl, step=L, unroll=8)
            def _(c):
                x16 = xv[pl.ds(c, L)]
                u = x16 * scale
                i16 = jnp.minimum(u.astype(jnp.int32), nk - 2)
                tt = u - i16.astype(jnp.float32)
                pw = lax.bitcast_convert_type(
                    plsc.load_gather(pv, [i16]), jnp.uint32)
                qw = lax.bitcast_convert_type(
                    plsc.load_gather(qv, [i16]), jnp.uint32)
                a, b = unpack_hi(pw), unpack_lo(pw)
                cc, dd = unpack_hi(qw), unpack_lo(qw)
                ov[pl.ds(c, L)] = a + tt * (b + tt * (cc + tt * dd))

            out_cpys.append(
                pltpu.async_copy(
                    ov.at[pl.ds(s * sl, sl)],
                    out_hbm.at[pl.ds(base + s * sl, sl)],
                    out_sems.at[s],
                )
            )
        for c in out_cpys:
            c.wait()

    return sc_spline(xs, y)


# speedup vs baseline: 1.0055x; 1.0055x over previous
import dataclasses
import functools

import jax
import jax.numpy as jnp
from jax import lax
from jax.experimental import pallas as pl
from jax.experimental.pallas import tpu as pltpu
from jax.experimental.pallas import tpu_sc as plsc

NC = 2
NS = 16
NW = NC * NS
L = 16
NSLICE = 2


def kernel(xs, t, y):
    del t
    n = xs.shape[0]
    nk = y.shape[0]
    chunk = n // NW
    scale = jnp.float32(nk - 1)

    mesh = plsc.VectorSubcoreMesh(core_axis_name="c", subcore_axis_name="s")

    cp = pltpu.CompilerParams()
    if "needs_layout_passes" in pltpu.CompilerParams.__dataclass_fields__:
        cp = dataclasses.replace(cp, needs_layout_passes=False)

    @functools.partial(
        pl.kernel,
        out_type=jax.ShapeDtypeStruct((n,), jnp.float32),
        mesh=mesh,
        compiler_params=cp,
        scratch_types=[
            pltpu.VMEM((chunk,), jnp.float32),
            pltpu.VMEM((chunk,), jnp.float32),
            pltpu.VMEM((nk,), jnp.float32),
            pltpu.VMEM((nk,), jnp.float32),
            pltpu.VMEM((nk,), jnp.float32),
            pltpu.VMEM((nk,), jnp.int32),
            pltpu.VMEM((nk,), jnp.int32),
            pltpu.SemaphoreType.DMA((NSLICE,)),
            pltpu.SemaphoreType.DMA((NSLICE,)),
            pltpu.SemaphoreType.DMA,
        ],
    )
    def sc_spline(xs_hbm, y_hbm, out_hbm, xv, ov, yv, dv, gv, pv, qv,
                  in_sems, out_sems, ysem):
        def rne16(v):
            u = lax.bitcast_convert_type(v, jnp.uint32)
            return (u + jnp.uint32(0x7FFF) + ((u >> 16) & jnp.uint32(1))) >> 16

        def unpack_hi(w):
            return lax.bitcast_convert_type(w & jnp.uint32(0xFFFF0000),
                                            jnp.float32)

        def unpack_lo(w):
            return lax.bitcast_convert_type(w << 16, jnp.float32)

        wid = lax.axis_index("s") * NC + lax.axis_index("c")
        base = wid * chunk
        sl = chunk // NSLICE
        in_cpys = [
            pltpu.async_copy(
                xs_hbm.at[pl.ds(base + s * sl, sl)],
                xv.at[pl.ds(s * sl, sl)],
                in_sems.at[s],
            )
            for s in range(NSLICE)
        ]
        pltpu.async_copy(y_hbm, yv, ysem).wait()

        @plsc.parallel_loop(0, nk, step=L, unroll=4)
        def _(c):
            idx = lax.iota(jnp.int32, L) + c
            y0 = plsc.load_gather(yv, [jnp.minimum(idx, nk - 1)])
            y1 = plsc.load_gather(yv, [jnp.minimum(idx + 1, nk - 1)])
            dv[pl.ds(c, L)] = y1 - y0

        @plsc.parallel_loop(0, nk, step=L, unroll=4)
        def _(c):
            idx = lax.iota(jnp.int32, L) + c
            lo = plsc.load_gather(dv, [jnp.maximum(idx - 1, 0)])
            hi = plsc.load_gather(dv, [jnp.minimum(idx, nk - 2)])
            gv[pl.ds(c, L)] = 0.5 * (lo + hi)

        @plsc.parallel_loop(0, nk, step=L, unroll=4)
        def _(c):
            idx = lax.iota(jnp.int32, L) + c
            dd = plsc.load_gather(dv, [jnp.minimum(idx, nk - 2)])
            gi = plsc.load_gather(gv, [idx])
            gi1 = plsc.load_gather(gv, [jnp.minimum(idx + 1, nk - 1)])
            y16 = yv[pl.ds(c, L)]
            cc = 3.0 * dd - 2.0 * gi - gi1
            qq = gi + gi1 - 2.0 * dd
            pv[pl.ds(c, L)] = lax.bitcast_convert_type(
                (rne16(y16) << 16) | rne16(gi), jnp.int32)
            qv[pl.ds(c, L)] = lax.bitcast_convert_type(
                (rne16(cc) << 16) | rne16(qq), jnp.int32)

        out_cpys = []
        for s in range(NSLICE):
            in_cpys[s].wait()

            @plsc.parallel_loop(s * sl, (s + 1) * sl, step=L, unroll=8)
            def _(c):
                x16 = xv[pl.ds(c, L)]
                u = x16 * scale
                i16 = jnp.minimum(u.astype(jnp.int32), nk - 2)
                tt = u - i16.astype(jnp.float32)
                pw = lax.bitcast_convert_type(
                    plsc.load_gather(pv, [i16]), jnp.uint32)
                qw = lax.bitcast_convert_type(
                    plsc.load_gather(qv, [i16]), jnp.uint32)
                a, b = unpack_hi(pw), unpack_lo(pw)
                cc, dd = unpack_hi(qw), unpack_lo(qw)
                ov[pl.ds(c, L)] = a + tt * (b + tt * (cc + tt * dd))

            out_cpys.append(
                pltpu.async_copy(
                    ov.at[pl.ds(s * sl, sl)],
                    out_hbm.at[pl.ds(base + s * sl, sl)],
                    out_sems.at[s],
                )
            )
        for c in out_cpys:
            c.wait()

    return sc_spline(xs, y)
